# baseline (speedup 1.0000x reference)
import functools

import jax
import jax.numpy as jnp
from jax.experimental import pallas as pl

_ALPHA = 0.2
_LOG2E = 1.4426950408889634
_BR = 256
_AUG = 128


def _aug(wh, dout):
    br = wh.shape[0]
    return jnp.concatenate(
        [wh, jnp.ones((br, 1), jnp.float32),
         jnp.zeros((br, _AUG - dout - 1), jnp.float32)],
        axis=1).astype(jnp.bfloat16)


def _exp_pair(logits):
    t = logits * jnp.float32(_LOG2E)
    return jnp.exp2(t), jnp.exp2(t * jnp.float32(_ALPHA))


def _proj1_body(x_ref, w1_ref, a1_ref, wh_refs, es_ref, edt_ref, *, nheads,
                dout):
    xb = x_ref[...]
    for h in range(nheads):
        wh = jnp.dot(xb, w1_ref[h], preferred_element_type=jnp.float32)
        wh_refs[h][...] = _aug(wh, dout)
        asrc = a1_ref[h:h + 1, :dout]
        adst = a1_ref[h:h + 1, dout:]
        es = jax.lax.dot_general(
            wh, asrc, (((1,), (1,)), ((), ())),
            preferred_element_type=jnp.float32)
        es_ref[:, 2 * h:2 * h + 1], es_ref[:, 2 * h + 1:2 * h + 2] = (
            _exp_pair(es))
        edt = jax.lax.dot_general(
            adst, wh, (((1,), (1,)), ((), ())),
            preferred_element_type=jnp.float32)
        edt_ref[2 * h:2 * h + 1, :], edt_ref[2 * h + 1:2 * h + 2, :] = (
            _exp_pair(edt))


def _attn_rows(adjf, wh_aug_ref, u_col, ua_col, v_row, va_row, dout):
    p = jnp.maximum(u_col * v_row, ua_col * va_row) * adjf
    wh_aug = wh_aug_ref[...]
    o_aug = jnp.dot(p.astype(jnp.bfloat16), wh_aug,
                    preferred_element_type=jnp.float32)
    o = o_aug[:, :dout]
    s = o_aug[:, dout:dout + 1]
    n = wh_aug.shape[0]
    colmean = jnp.sum(wh_aug[:, :dout].astype(jnp.float32), axis=0,
                      keepdims=True) * (1.0 / n)
    return jnp.where(s > 0, o / jnp.where(s > 0, s, 1.0), colmean)


def _att1_body(adj_ref, wh_refs, es_ref, edt_ref, w2_ref, a2_ref, wh2o_refs,
               es2_ref, edt2_ref, *, nheads, nouts, dout, nclass):
    adjf = adj_ref[...].astype(jnp.float32)
    cols = []
    for h in range(nheads):
        oh = _attn_rows(adjf, wh_refs[h],
                        es_ref[:, 2 * h:2 * h + 1],
                        es_ref[:, 2 * h + 1:2 * h + 2],
                        edt_ref[2 * h:2 * h + 1, :],
                        edt_ref[2 * h + 1:2 * h + 2, :], dout)
        cols.append(jnp.where(oh > 0, oh, jnp.exp(jnp.minimum(oh, 0.0)) - 1.0))
    h1b = jnp.concatenate(cols, axis=1)
    for j in range(nouts):
        whj = jnp.dot(h1b, w2_ref[j], preferred_element_type=jnp.float32)
        wh2o_refs[j][...] = _aug(whj, nclass)
        asrc = a2_ref[j:j + 1, :nclass]
        adst = a2_ref[j:j + 1, nclass:]
        es2 = jax.lax.dot_general(
            whj, asrc, (((1,), (1,)), ((), ())),
            preferred_element_type=jnp.float32)
        es2_ref[:, 2 * j:2 * j + 1], es2_ref[:, 2 * j + 1:2 * j + 2] = (
            _exp_pair(es2))
        edt2 = jax.lax.dot_general(
            adst, whj, (((1,), (1,)), ((), ())),
            preferred_element_type=jnp.float32)
        edt2_ref[2 * j:2 * j + 1, :], edt2_ref[2 * j + 1:2 * j + 2, :] = (
            _exp_pair(edt2))


def _att2_body(adj_ref, wh_refs, es_ref, edt_ref, out_ref, *, nouts, nclass):
    adjf = adj_ref[...].astype(jnp.float32)
    acc = None
    for j in range(nouts):
        oj = _attn_rows(adjf, wh_refs[j],
                        es_ref[:, 2 * j:2 * j + 1],
                        es_ref[:, 2 * j + 1:2 * j + 2],
                        edt_ref[2 * j:2 * j + 1, :],
                        edt_ref[2 * j + 1:2 * j + 2, :], nclass)
        acc = oj if acc is None else acc + oj
    o = acc * (1.0 / nouts)
    m = jnp.max(o, axis=1, keepdims=True)
    lse = jnp.log(jnp.sum(jnp.exp(o - m), axis=1, keepdims=True)) + m
    out_ref[...] = o - lse


def kernel(x, adj, W1, a1, W2, a2):
    n, nfeat = x.shape
    nheads, _, dout = W1.shape
    nouts, nhid_tot, nclass = W2.shape
    br = _BR
    grid = (n // br,)

    full = lambda shape: pl.BlockSpec(shape, lambda i: (0,) * len(shape))
    rows = lambda shape: pl.BlockSpec((br,) + shape[1:], lambda i: (i,) + (0,) * (len(shape) - 1))
    colsb = lambda lead: pl.BlockSpec((lead, br), lambda i: (0, i))

    proj1 = pl.pallas_call(
        functools.partial(_proj1_wrap, nheads=nheads, dout=dout),
        grid=grid,
        in_specs=[rows((n, nfeat)), full((nheads, nfeat, dout)),
                  full((nheads, 2 * dout))],
        out_specs=tuple([rows((n, _AUG))] * nheads
                        + [rows((n, 2 * nheads)), colsb(2 * nheads)]),
        out_shape=tuple(
            [jax.ShapeDtypeStruct((n, _AUG), jnp.bfloat16)] * nheads
            + [jax.ShapeDtypeStruct((n, 2 * nheads), jnp.float32),
               jax.ShapeDtypeStruct((2 * nheads, n), jnp.float32)]),
    )
    *wh1, es1, edt1 = proj1(x, W1, a1)

    att1 = pl.pallas_call(
        functools.partial(_att1_wrap, nheads=nheads, nouts=nouts, dout=dout,
                          nclass=nclass),
        grid=grid,
        in_specs=[rows((n, n))] + [full((n, _AUG))] * nheads
                 + [rows((n, 2 * nheads)), full((2 * nheads, n)),
                    full((nouts, nhid_tot, nclass)), full((nouts, 2 * nclass))],
        out_specs=tuple([rows((n, _AUG))] * nouts
                        + [rows((n, 2 * nouts)), colsb(2 * nouts)]),
        out_shape=tuple(
            [jax.ShapeDtypeStruct((n, _AUG), jnp.bfloat16)] * nouts
            + [jax.ShapeDtypeStruct((n, 2 * nouts), jnp.float32),
               jax.ShapeDtypeStruct((2 * nouts, n), jnp.float32)]),
    )
    *wh2, es2, edt2 = att1(adj, *wh1, es1, edt1, W2, a2)

    att2 = pl.pallas_call(
        functools.partial(_att2_wrap, nouts=nouts, nclass=nclass),
        grid=grid,
        in_specs=[rows((n, n))] + [full((n, _AUG))] * nouts
                 + [rows((n, 2 * nouts)), full((2 * nouts, n))],
        out_specs=rows((n, nclass)),
        out_shape=jax.ShapeDtypeStruct((n, nclass), jnp.float32),
    )
    return att2(adj, *wh2, es2, edt2)


def _proj1_wrap(x_ref, w1_ref, a1_ref, *out_refs, nheads, dout):
    _proj1_body(x_ref, w1_ref, a1_ref, out_refs[:nheads], out_refs[nheads],
                out_refs[nheads + 1], nheads=nheads, dout=dout)


def _att1_wrap(adj_ref, *refs, nheads, nouts, dout, nclass):
    wh_refs = refs[:nheads]
    es_ref, edt_ref, w2_ref, a2_ref = refs[nheads:nheads + 4]
    out_refs = refs[nheads + 4:]
    _att1_body(adj_ref, wh_refs, es_ref, edt_ref, w2_ref, a2_ref,
               out_refs[:nouts], out_refs[nouts], out_refs[nouts + 1],
               nheads=nheads, nouts=nouts, dout=dout, nclass=nclass)


def _att2_wrap(adj_ref, *refs, nouts, nclass):
    wh_refs = refs[:nouts]
    es_ref, edt_ref, out_ref = refs[nouts:]
    _att2_body(adj_ref, wh_refs, es_ref, edt_ref, out_ref, nouts=nouts,
               nclass=nclass)


# speedup vs baseline: 1.3175x; 1.3175x over previous
import functools

import jax
import jax.numpy as jnp
from jax.experimental import pallas as pl

_ALPHA = 0.2
_LOG2E = 1.4426950408889634
_BR = 256
_AUG = 128


def _aug(wh, dout):
    br = wh.shape[0]
    return jnp.concatenate(
        [wh, jnp.ones((br, 1), jnp.float32),
         jnp.zeros((br, _AUG - dout - 1), jnp.float32)],
        axis=1).astype(jnp.bfloat16)


def _exp_pair(logits):
    t = logits * jnp.float32(_LOG2E)
    return (jnp.exp2(t).astype(jnp.bfloat16),
            jnp.exp2(t * jnp.float32(_ALPHA)).astype(jnp.bfloat16))


def _proj1_body(x_ref, w1_ref, a1_ref, wh_refs, es_ref, edt_ref, cs_ref, *,
                nheads, dout):
    xb = x_ref[...]
    contribs = []
    for h in range(nheads):
        wh = jnp.dot(xb, w1_ref[h], preferred_element_type=jnp.float32)
        wh_refs[h][...] = _aug(wh, dout)
        contribs.append(jnp.sum(wh, axis=0, keepdims=True))
        asrc = a1_ref[h:h + 1, :dout]
        adst = a1_ref[h:h + 1, dout:]
        es = jax.lax.dot_general(
            wh, asrc, (((1,), (1,)), ((), ())),
            preferred_element_type=jnp.float32)
        es_ref[:, 2 * h:2 * h + 1], es_ref[:, 2 * h + 1:2 * h + 2] = (
            _exp_pair(es))
        edt = jax.lax.dot_general(
            adst, wh, (((1,), (1,)), ((), ())),
            preferred_element_type=jnp.float32)
        edt_ref[2 * h:2 * h + 1, :], edt_ref[2 * h + 1:2 * h + 2, :] = (
            _exp_pair(edt))
    _accum_colsum(cs_ref, jnp.concatenate(contribs, axis=0))


def _accum_colsum(cs_ref, contrib):
    @pl.when(pl.program_id(0) == 0)
    def _():
        cs_ref[...] = contrib

    @pl.when(pl.program_id(0) != 0)
    def _():
        cs_ref[...] = cs_ref[...] + contrib


def _attn_rows(adjf, wh_aug_ref, u_col, ua_col, v_row, va_row, colmean):
    p = jnp.maximum(u_col * v_row, ua_col * va_row) * adjf
    dout = colmean.shape[1]
    o_aug = jnp.dot(p, wh_aug_ref[...], preferred_element_type=jnp.float32)
    o = o_aug[:, :dout]
    s = o_aug[:, dout:dout + 1]
    return jnp.where(s > 0, o / jnp.where(s > 0, s, 1.0), colmean)


def _att1_body(adj_ref, wh_refs, es_ref, edt_ref, w2_ref, a2_ref, cs1_ref,
               wh2o_refs, es2_ref, edt2_ref, cs2_ref, *, nheads, nouts, dout,
               nclass):
    n = adj_ref.shape[1]
    adjf = adj_ref[...].astype(jnp.bfloat16)
    cols = []
    for h in range(nheads):
        oh = _attn_rows(adjf, wh_refs[h],
                        es_ref[:, 2 * h:2 * h + 1],
                        es_ref[:, 2 * h + 1:2 * h + 2],
                        edt_ref[2 * h:2 * h + 1, :],
                        edt_ref[2 * h + 1:2 * h + 2, :],
                        cs1_ref[h:h + 1, :] * (1.0 / n))
        cols.append(jnp.where(oh > 0, oh, jnp.exp(jnp.minimum(oh, 0.0)) - 1.0))
    h1b = jnp.concatenate(cols, axis=1)
    contribs = []
    for j in range(nouts):
        whj = jnp.dot(h1b, w2_ref[j], preferred_element_type=jnp.float32)
        wh2o_refs[j][...] = _aug(whj, nclass)
        contribs.append(jnp.sum(whj, axis=0, keepdims=True))
        asrc = a2_ref[j:j + 1, :nclass]
        adst = a2_ref[j:j + 1, nclass:]
        es2 = jax.lax.dot_general(
            whj, asrc, (((1,), (1,)), ((), ())),
            preferred_element_type=jnp.float32)
        es2_ref[:, 2 * j:2 * j + 1], es2_ref[:, 2 * j + 1:2 * j + 2] = (
            _exp_pair(es2))
        edt2 = jax.lax.dot_general(
            adst, whj, (((1,), (1,)), ((), ())),
            preferred_element_type=jnp.float32)
        edt2_ref[2 * j:2 * j + 1, :], edt2_ref[2 * j + 1:2 * j + 2, :] = (
            _exp_pair(edt2))
    _accum_colsum(cs2_ref, jnp.concatenate(contribs, axis=0))


def _att2_body(adj_ref, wh_refs, es_ref, edt_ref, cs2_ref, out_ref, *, nouts,
               nclass):
    n = adj_ref.shape[1]
    adjf = adj_ref[...].astype(jnp.bfloat16)
    acc = None
    for j in range(nouts):
        oj = _attn_rows(adjf, wh_refs[j],
                        es_ref[:, 2 * j:2 * j + 1],
                        es_ref[:, 2 * j + 1:2 * j + 2],
                        edt_ref[2 * j:2 * j + 1, :],
                        edt_ref[2 * j + 1:2 * j + 2, :],
                        cs2_ref[j:j + 1, :] * (1.0 / n))
        acc = oj if acc is None else acc + oj
    o = acc * (1.0 / nouts)
    m = jnp.max(o, axis=1, keepdims=True)
    lse = jnp.log(jnp.sum(jnp.exp(o - m), axis=1, keepdims=True)) + m
    out_ref[...] = o - lse


def kernel(x, adj, W1, a1, W2, a2):
    n, nfeat = x.shape
    nheads, _, dout = W1.shape
    nouts, nhid_tot, nclass = W2.shape
    br = _BR
    grid = (n // br,)

    full = lambda shape: pl.BlockSpec(shape, lambda i: (0,) * len(shape))
    rows = lambda shape: pl.BlockSpec((br,) + shape[1:], lambda i: (i,) + (0,) * (len(shape) - 1))
    colsb = lambda lead: pl.BlockSpec((lead, br), lambda i: (0, i))

    proj1 = pl.pallas_call(
        functools.partial(_proj1_wrap, nheads=nheads, dout=dout),
        grid=grid,
        in_specs=[rows((n, nfeat)), full((nheads, nfeat, dout)),
                  full((nheads, 2 * dout))],
        out_specs=tuple([rows((n, _AUG))] * nheads
                        + [rows((n, 2 * nheads)), colsb(2 * nheads),
                           full((nheads, dout))]),
        out_shape=tuple(
            [jax.ShapeDtypeStruct((n, _AUG), jnp.bfloat16)] * nheads
            + [jax.ShapeDtypeStruct((n, 2 * nheads), jnp.bfloat16),
               jax.ShapeDtypeStruct((2 * nheads, n), jnp.bfloat16),
               jax.ShapeDtypeStruct((nheads, dout), jnp.float32)]),
    )
    *wh1, es1, edt1, cs1 = proj1(x, W1, a1)

    att1 = pl.pallas_call(
        functools.partial(_att1_wrap, nheads=nheads, nouts=nouts, dout=dout,
                          nclass=nclass),
        grid=grid,
        in_specs=[rows((n, n))] + [full((n, _AUG))] * nheads
                 + [rows((n, 2 * nheads)), full((2 * nheads, n)),
                    full((nouts, nhid_tot, nclass)), full((nouts, 2 * nclass)),
                    full((nheads, dout))],
        out_specs=tuple([rows((n, _AUG))] * nouts
                        + [rows((n, 2 * nouts)), colsb(2 * nouts),
                           full((nouts, nclass))]),
        out_shape=tuple(
            [jax.ShapeDtypeStruct((n, _AUG), jnp.bfloat16)] * nouts
            + [jax.ShapeDtypeStruct((n, 2 * nouts), jnp.bfloat16),
               jax.ShapeDtypeStruct((2 * nouts, n), jnp.bfloat16),
               jax.ShapeDtypeStruct((nouts, nclass), jnp.float32)]),
    )
    *wh2, es2, edt2, cs2 = att1(adj, *wh1, es1, edt1, W2, a2, cs1)

    att2 = pl.pallas_call(
        functools.partial(_att2_wrap, nouts=nouts, nclass=nclass),
        grid=grid,
        in_specs=[rows((n, n))] + [full((n, _AUG))] * nouts
                 + [rows((n, 2 * nouts)), full((2 * nouts, n)),
                    full((nouts, nclass))],
        out_specs=rows((n, nclass)),
        out_shape=jax.ShapeDtypeStruct((n, nclass), jnp.float32),
    )
    return att2(adj, *wh2, es2, edt2, cs2)


def _proj1_wrap(x_ref, w1_ref, a1_ref, *out_refs, nheads, dout):
    _proj1_body(x_ref, w1_ref, a1_ref, out_refs[:nheads], out_refs[nheads],
                out_refs[nheads + 1], out_refs[nheads + 2], nheads=nheads,
                dout=dout)


def _att1_wrap(adj_ref, *refs, nheads, nouts, dout, nclass):
    wh_refs = refs[:nheads]
    es_ref, edt_ref, w2_ref, a2_ref, cs1_ref = refs[nheads:nheads + 5]
    out_refs = refs[nheads + 5:]
    _att1_body(adj_ref, wh_refs, es_ref, edt_ref, w2_ref, a2_ref, cs1_ref,
               out_refs[:nouts], out_refs[nouts], out_refs[nouts + 1],
               out_refs[nouts + 2], nheads=nheads, nouts=nouts, dout=dout,
               nclass=nclass)


def _att2_wrap(adj_ref, *refs, nouts, nclass):
    wh_refs = refs[:nouts]
    es_ref, edt_ref, cs2_ref, out_ref = refs[nouts:]
    _att2_body(adj_ref, wh_refs, es_ref, edt_ref, cs2_ref, out_ref,
               nouts=nouts, nclass=nclass)


# speedup vs baseline: 1.5738x; 1.1945x over previous
import functools

import jax
import jax.numpy as jnp
from jax.experimental import pallas as pl

_ALPHA = 0.2
_LOG2E = 1.4426950408889634
_BR = 512
_AUG = 128


def _aug(wh, dout):
    br = wh.shape[0]
    return jnp.concatenate(
        [wh, jnp.ones((br, 1), jnp.float32),
         jnp.zeros((br, _AUG - dout - 1), jnp.float32)],
        axis=1).astype(jnp.bfloat16)


def _exp_pair(logits):
    t = logits * jnp.float32(_LOG2E)
    return (jnp.exp2(t).astype(jnp.bfloat16),
            jnp.exp2(t * jnp.float32(_ALPHA)).astype(jnp.bfloat16))


def _exp_w(logits):
    return jnp.exp2(
        logits * jnp.float32(_LOG2E * (_ALPHA - 1.0))).astype(jnp.bfloat16)


def _proj1_body(x_ref, w1_ref, a1_ref, wh_refs, es_ref, edt_ref, cs_ref, *,
                nheads, dout):
    xb = x_ref[...]
    contribs = []
    for h in range(nheads):
        wh = jnp.dot(xb, w1_ref[h], preferred_element_type=jnp.float32)
        wh_refs[h][...] = _aug(wh, dout)
        contribs.append(jnp.sum(wh, axis=0, keepdims=True))
        asrc = a1_ref[h:h + 1, :dout]
        adst = a1_ref[h:h + 1, dout:]
        es = jax.lax.dot_general(
            wh, asrc, (((1,), (1,)), ((), ())),
            preferred_element_type=jnp.float32)
        es_ref[:, h:h + 1] = _exp_w(es)
        edt = jax.lax.dot_general(
            adst, wh, (((1,), (1,)), ((), ())),
            preferred_element_type=jnp.float32)
        edt_ref[2 * h:2 * h + 1, :], edt_ref[2 * h + 1:2 * h + 2, :] = (
            _exp_pair(edt))
    _accum_colsum(cs_ref, jnp.concatenate(contribs, axis=0))


def _accum_colsum(cs_ref, contrib):
    @pl.when(pl.program_id(0) == 0)
    def _():
        cs_ref[...] = contrib

    @pl.when(pl.program_id(0) != 0)
    def _():
        cs_ref[...] = cs_ref[...] + contrib


def _attn_rows(adjf, wh_aug_ref, w_col, v_row, va_row, colmean):
    p = jnp.maximum(v_row, w_col * va_row) * adjf
    dout = colmean.shape[1]
    o_aug = jnp.dot(p, wh_aug_ref[...], preferred_element_type=jnp.float32)
    o = o_aug[:, :dout]
    s = o_aug[:, dout:dout + 1]
    return jnp.where(s > 0, o / jnp.where(s > 0, s, 1.0), colmean)


def _att1_body(adj_ref, wh_refs, es_ref, edt_ref, w2_ref, a2_ref, cs1_ref,
               wh2o_refs, es2_ref, edt2_ref, cs2_ref, adjb_ref, *, nheads,
               nouts, dout, nclass):
    n = adj_ref.shape[1]
    adjf = adj_ref[...].astype(jnp.bfloat16)
    adjb_ref[...] = adjf
    cols = []
    for h in range(nheads):
        oh = _attn_rows(adjf, wh_refs[h],
                        es_ref[:, h:h + 1],
                        edt_ref[2 * h:2 * h + 1, :],
                        edt_ref[2 * h + 1:2 * h + 2, :],
                        cs1_ref[h:h + 1, :] * (1.0 / n))
        cols.append(jnp.where(oh > 0, oh, jnp.exp(jnp.minimum(oh, 0.0)) - 1.0))
    h1b = jnp.concatenate(cols, axis=1)
    contribs = []
    for j in range(nouts):
        whj = jnp.dot(h1b, w2_ref[j], preferred_element_type=jnp.float32)
        wh2o_refs[j][...] = _aug(whj, nclass)
        contribs.append(jnp.sum(whj, axis=0, keepdims=True))
        asrc = a2_ref[j:j + 1, :nclass]
        adst = a2_ref[j:j + 1, nclass:]
        es2 = jax.lax.dot_general(
            whj, asrc, (((1,), (1,)), ((), ())),
            preferred_element_type=jnp.float32)
        es2_ref[:, j:j + 1] = _exp_w(es2)
        edt2 = jax.lax.dot_general(
            adst, whj, (((1,), (1,)), ((), ())),
            preferred_element_type=jnp.float32)
        edt2_ref[2 * j:2 * j + 1, :], edt2_ref[2 * j + 1:2 * j + 2, :] = (
            _exp_pair(edt2))
    _accum_colsum(cs2_ref, jnp.concatenate(contribs, axis=0))


def _att2_body(adjb_ref, wh_refs, es_ref, edt_ref, cs2_ref, out_ref, *, nouts,
               nclass):
    n = adjb_ref.shape[1]
    adjf = adjb_ref[...]
    acc = None
    for j in range(nouts):
        oj = _attn_rows(adjf, wh_refs[j],
                        es_ref[:, j:j + 1],
                        edt_ref[2 * j:2 * j + 1, :],
                        edt_ref[2 * j + 1:2 * j + 2, :],
                        cs2_ref[j:j + 1, :] * (1.0 / n))
        acc = oj if acc is None else acc + oj
    o = acc * (1.0 / nouts)
    m = jnp.max(o, axis=1, keepdims=True)
    lse = jnp.log(jnp.sum(jnp.exp(o - m), axis=1, keepdims=True)) + m
    out_ref[...] = o - lse


def kernel(x, adj, W1, a1, W2, a2):
    n, nfeat = x.shape
    nheads, _, dout = W1.shape
    nouts, nhid_tot, nclass = W2.shape
    br = _BR
    grid = (n // br,)

    full = lambda shape: pl.BlockSpec(shape, lambda i: (0,) * len(shape))
    rows = lambda shape: pl.BlockSpec((br,) + shape[1:], lambda i: (i,) + (0,) * (len(shape) - 1))
    colsb = lambda lead: pl.BlockSpec((lead, br), lambda i: (0, i))

    proj1 = pl.pallas_call(
        functools.partial(_proj1_wrap, nheads=nheads, dout=dout),
        grid=grid,
        in_specs=[rows((n, nfeat)), full((nheads, nfeat, dout)),
                  full((nheads, 2 * dout))],
        out_specs=tuple([rows((n, _AUG))] * nheads
                        + [rows((n, nheads)), colsb(2 * nheads),
                           full((nheads, dout))]),
        out_shape=tuple(
            [jax.ShapeDtypeStruct((n, _AUG), jnp.bfloat16)] * nheads
            + [jax.ShapeDtypeStruct((n, nheads), jnp.bfloat16),
               jax.ShapeDtypeStruct((2 * nheads, n), jnp.bfloat16),
               jax.ShapeDtypeStruct((nheads, dout), jnp.float32)]),
    )
    *wh1, es1, edt1, cs1 = proj1(x, W1, a1)

    att1 = pl.pallas_call(
        functools.partial(_att1_wrap, nheads=nheads, nouts=nouts, dout=dout,
                          nclass=nclass),
        grid=grid,
        in_specs=[rows((n, n))] + [full((n, _AUG))] * nheads
                 + [rows((n, nheads)), full((2 * nheads, n)),
                    full((nouts, nhid_tot, nclass)), full((nouts, 2 * nclass)),
                    full((nheads, dout))],
        out_specs=tuple([rows((n, _AUG))] * nouts
                        + [rows((n, nouts)), colsb(2 * nouts),
                           full((nouts, nclass)), rows((n, n))]),
        out_shape=tuple(
            [jax.ShapeDtypeStruct((n, _AUG), jnp.bfloat16)] * nouts
            + [jax.ShapeDtypeStruct((n, nouts), jnp.bfloat16),
               jax.ShapeDtypeStruct((2 * nouts, n), jnp.bfloat16),
               jax.ShapeDtypeStruct((nouts, nclass), jnp.float32),
               jax.ShapeDtypeStruct((n, n), jnp.bfloat16)]),
    )
    *wh2, es2, edt2, cs2, adjb = att1(adj, *wh1, es1, edt1, W2, a2, cs1)

    att2 = pl.pallas_call(
        functools.partial(_att2_wrap, nouts=nouts, nclass=nclass),
        grid=grid,
        in_specs=[rows((n, n))] + [full((n, _AUG))] * nouts
                 + [rows((n, nouts)), full((2 * nouts, n)),
                    full((nouts, nclass))],
        out_specs=rows((n, nclass)),
        out_shape=jax.ShapeDtypeStruct((n, nclass), jnp.float32),
    )
    return att2(adjb, *wh2, es2, edt2, cs2)


def _proj1_wrap(x_ref, w1_ref, a1_ref, *out_refs, nheads, dout):
    _proj1_body(x_ref, w1_ref, a1_ref, out_refs[:nheads], out_refs[nheads],
                out_refs[nheads + 1], out_refs[nheads + 2], nheads=nheads,
                dout=dout)


def _att1_wrap(adj_ref, *refs, nheads, nouts, dout, nclass):
    wh_refs = refs[:nheads]
    es_ref, edt_ref, w2_ref, a2_ref, cs1_ref = refs[nheads:nheads + 5]
    out_refs = refs[nheads + 5:]
    _att1_body(adj_ref, wh_refs, es_ref, edt_ref, w2_ref, a2_ref, cs1_ref,
               out_refs[:nouts], out_refs[nouts], out_refs[nouts + 1],
               out_refs[nouts + 2], out_refs[nouts + 3], nheads=nheads,
               nouts=nouts, dout=dout, nclass=nclass)


def _att2_wrap(adj_ref, *refs, nouts, nclass):
    wh_refs = refs[:nouts]
    es_ref, edt_ref, cs2_ref, out_ref = refs[nouts:]
    _att2_body(adj_ref, wh_refs, es_ref, edt_ref, cs2_ref, out_ref,
               nouts=nouts, nclass=nclass)


# speedup vs baseline: 1.5894x; 1.0099x over previous
import functools

import jax
import jax.numpy as jnp
from jax.experimental import pallas as pl

_ALPHA = 0.2
_LOG2E = 1.4426950408889634
_BR1 = 1024
_BR_ATT1 = 512
_AUG = 128


def _aug(wh, dout):
    br = wh.shape[0]
    return jnp.concatenate(
        [wh, jnp.ones((br, 1), jnp.float32),
         jnp.zeros((br, _AUG - dout - 1), jnp.float32)],
        axis=1).astype(jnp.bfloat16)


def _exp_pair(logits):
    t = logits * jnp.float32(_LOG2E)
    return (jnp.exp2(t).astype(jnp.bfloat16),
            jnp.exp2(t * jnp.float32(_ALPHA)).astype(jnp.bfloat16))


def _exp_w(logits):
    return jnp.exp2(
        logits * jnp.float32(_LOG2E * (_ALPHA - 1.0))).astype(jnp.bfloat16)


def _proj1_body(x_ref, w1_ref, a1_ref, wh_refs, es_ref, edt_ref, cs_ref, *,
                nheads, dout):
    xb = x_ref[...]
    contribs = []
    for h in range(nheads):
        wh = jnp.dot(xb, w1_ref[h], preferred_element_type=jnp.float32)
        wh_refs[h][...] = _aug(wh, dout)
        contribs.append(jnp.sum(wh, axis=0, keepdims=True))
        asrc = a1_ref[h:h + 1, :dout]
        adst = a1_ref[h:h + 1, dout:]
        es = jax.lax.dot_general(
            wh, asrc, (((1,), (1,)), ((), ())),
            preferred_element_type=jnp.float32)
        es_ref[:, h:h + 1] = _exp_w(es)
        edt = jax.lax.dot_general(
            adst, wh, (((1,), (1,)), ((), ())),
            preferred_element_type=jnp.float32)
        edt_ref[2 * h:2 * h + 1, :], edt_ref[2 * h + 1:2 * h + 2, :] = (
            _exp_pair(edt))
    _accum_colsum(cs_ref, jnp.concatenate(contribs, axis=0))


def _accum_colsum(cs_ref, contrib):
    @pl.when(pl.program_id(0) == 0)
    def _():
        cs_ref[...] = contrib

    @pl.when(pl.program_id(0) != 0)
    def _():
        cs_ref[...] = cs_ref[...] + contrib


def _attn_rows(adjf, wh_aug_ref, w_col, v_row, va_row, colmean):
    p = jnp.maximum(v_row, w_col * va_row) * adjf
    dout = colmean.shape[1]
    o_aug = jnp.dot(p, wh_aug_ref[...], preferred_element_type=jnp.float32)
    o = o_aug[:, :dout]
    s = o_aug[:, dout:dout + 1]
    return jnp.where(s > 0, o / jnp.where(s > 0, s, 1.0), colmean)


def _att1_body(adj_ref, wh_refs, es_ref, edt_ref, w2_ref, a2_ref, cs1_ref,
               wh2o_refs, es2_ref, edt2_ref, cs2_ref, adjb_ref, *, nheads,
               nouts, dout, nclass):
    n = adj_ref.shape[1]
    adjf = adj_ref[...].astype(jnp.bfloat16)
    adjb_ref[...] = adjf
    cols = []
    for h in range(nheads):
        oh = _attn_rows(adjf, wh_refs[h],
                        es_ref[:, h:h + 1],
                        edt_ref[2 * h:2 * h + 1, :],
                        edt_ref[2 * h + 1:2 * h + 2, :],
                        cs1_ref[h:h + 1, :] * (1.0 / n))
        cols.append(jnp.where(oh > 0, oh, jnp.exp(jnp.minimum(oh, 0.0)) - 1.0))
    h1b = jnp.concatenate(cols, axis=1)
    contribs = []
    for j in range(nouts):
        whj = jnp.dot(h1b, w2_ref[j], preferred_element_type=jnp.float32)
        wh2o_refs[j][...] = _aug(whj, nclass)
        contribs.append(jnp.sum(whj, axis=0, keepdims=True))
        asrc = a2_ref[j:j + 1, :nclass]
        adst = a2_ref[j:j + 1, nclass:]
        es2 = jax.lax.dot_general(
            whj, asrc, (((1,), (1,)), ((), ())),
            preferred_element_type=jnp.float32)
        es2_ref[:, j:j + 1] = _exp_w(es2)
        edt2 = jax.lax.dot_general(
            adst, whj, (((1,), (1,)), ((), ())),
            preferred_element_type=jnp.float32)
        edt2_ref[2 * j:2 * j + 1, :], edt2_ref[2 * j + 1:2 * j + 2, :] = (
            _exp_pair(edt2))
    _accum_colsum(cs2_ref, jnp.concatenate(contribs, axis=0))


def _att2_body(adjb_ref, wh_refs, es_ref, edt_ref, cs2_ref, out_ref, *, nouts,
               nclass):
    n = adjb_ref.shape[1]
    adjf = adjb_ref[...]
    acc = None
    for j in range(nouts):
        oj = _attn_rows(adjf, wh_refs[j],
                        es_ref[:, j:j + 1],
                        edt_ref[2 * j:2 * j + 1, :],
                        edt_ref[2 * j + 1:2 * j + 2, :],
                        cs2_ref[j:j + 1, :] * (1.0 / n))
        acc = oj if acc is None else acc + oj
    o = acc * (1.0 / nouts)
    m = jnp.max(o, axis=1, keepdims=True)
    lse = jnp.log(jnp.sum(jnp.exp(o - m), axis=1, keepdims=True)) + m
    out_ref[...] = o - lse


def kernel(x, adj, W1, a1, W2, a2):
    n, nfeat = x.shape
    nheads, _, dout = W1.shape
    nouts, nhid_tot, nclass = W2.shape
    full = lambda shape: pl.BlockSpec(shape, lambda i: (0,) * len(shape))
    rows = lambda br, shape: pl.BlockSpec(
        (br,) + shape[1:], lambda i: (i,) + (0,) * (len(shape) - 1))
    colsb = lambda br, lead: pl.BlockSpec((lead, br), lambda i: (0, i))

    br = min(_BR1, n)
    proj1 = pl.pallas_call(
        functools.partial(_proj1_wrap, nheads=nheads, dout=dout),
        grid=(n // br,),
        in_specs=[rows(br, (n, nfeat)), full((nheads, nfeat, dout)),
                  full((nheads, 2 * dout))],
        out_specs=tuple([rows(br, (n, _AUG))] * nheads
                        + [rows(br, (n, nheads)), colsb(br, 2 * nheads),
                           full((nheads, dout))]),
        out_shape=tuple(
            [jax.ShapeDtypeStruct((n, _AUG), jnp.bfloat16)] * nheads
            + [jax.ShapeDtypeStruct((n, nheads), jnp.bfloat16),
               jax.ShapeDtypeStruct((2 * nheads, n), jnp.bfloat16),
               jax.ShapeDtypeStruct((nheads, dout), jnp.float32)]),
    )
    *wh1, es1, edt1, cs1 = proj1(x, W1, a1)

    br = min(_BR_ATT1, n)
    att1 = pl.pallas_call(
        functools.partial(_att1_wrap, nheads=nheads, nouts=nouts, dout=dout,
                          nclass=nclass),
        grid=(n // br,),
        in_specs=[rows(br, (n, n))] + [full((n, _AUG))] * nheads
                 + [rows(br, (n, nheads)), full((2 * nheads, n)),
                    full((nouts, nhid_tot, nclass)), full((nouts, 2 * nclass)),
                    full((nheads, dout))],
        out_specs=tuple([rows(br, (n, _AUG))] * nouts
                        + [rows(br, (n, nouts)), colsb(br, 2 * nouts),
                           full((nouts, nclass)), rows(br, (n, n))]),
        out_shape=tuple(
            [jax.ShapeDtypeStruct((n, _AUG), jnp.bfloat16)] * nouts
            + [jax.ShapeDtypeStruct((n, nouts), jnp.bfloat16),
               jax.ShapeDtypeStruct((2 * nouts, n), jnp.bfloat16),
               jax.ShapeDtypeStruct((nouts, nclass), jnp.float32),
               jax.ShapeDtypeStruct((n, n), jnp.bfloat16)]),
    )
    *wh2, es2, edt2, cs2, adjb = att1(adj, *wh1, es1, edt1, W2, a2, cs1)

    br = min(_BR1, n)
    att2 = pl.pallas_call(
        functools.partial(_att2_wrap, nouts=nouts, nclass=nclass),
        grid=(n // br,),
        in_specs=[rows(br, (n, n))] + [full((n, _AUG))] * nouts
                 + [rows(br, (n, nouts)), full((2 * nouts, n)),
                    full((nouts, nclass))],
        out_specs=rows(br, (n, nclass)),
        out_shape=jax.ShapeDtypeStruct((n, nclass), jnp.float32),
    )
    return att2(adjb, *wh2, es2, edt2, cs2)


def _proj1_wrap(x_ref, w1_ref, a1_ref, *out_refs, nheads, dout):
    _proj1_body(x_ref, w1_ref, a1_ref, out_refs[:nheads], out_refs[nheads],
                out_refs[nheads + 1], out_refs[nheads + 2], nheads=nheads,
                dout=dout)


def _att1_wrap(adj_ref, *refs, nheads, nouts, dout, nclass):
    wh_refs = refs[:nheads]
    es_ref, edt_ref, w2_ref, a2_ref, cs1_ref = refs[nheads:nheads + 5]
    out_refs = refs[nheads + 5:]
    _att1_body(adj_ref, wh_refs, es_ref, edt_ref, w2_ref, a2_ref, cs1_ref,
               out_refs[:nouts], out_refs[nouts], out_refs[nouts + 1],
               out_refs[nouts + 2], out_refs[nouts + 3], nheads=nheads,
               nouts=nouts, dout=dout, nclass=nclass)


def _att2_wrap(adj_ref, *refs, nouts, nclass):
    wh_refs = refs[:nouts]
    es_ref, edt_ref, cs2_ref, out_ref = refs[nouts:]
    _att2_body(adj_ref, wh_refs, es_ref, edt_ref, cs2_ref, out_ref,
               nouts=nouts, nclass=nclass)
